# baseline (speedup 1.0000x reference)
import functools

import jax
import jax.numpy as jnp
from jax import lax
from jax.experimental import pallas as pl
from jax.experimental.pallas import tpu as pltpu
from jax.experimental.pallas import tpu_sc as plsc

N_NODES = 10000
N_EDGES = 320000
D = 128
N_LAYERS = 8

NC = 2
NS = 16
NW = NC * NS
K = 128
NB = 80
KING = 8
NCH = NB // KING + 2
EPT_PAD = NB * K
E_PAD = EPT_PAD * NW
ACC_ROWS = 10240
ROWS_PER_TILE = ACC_ROWS // NS


@functools.cache
def _make_sc_scatter():
    return functools.partial(
        pl.kernel,
        mesh=plsc.VectorSubcoreMesh(core_axis_name="c", subcore_axis_name="s"),
        out_type=jax.ShapeDtypeStruct((NW, ROWS_PER_TILE, D), jnp.float32),
        scratch_types=[
            pltpu.VMEM((NB + 2, K), jnp.int32),
            pltpu.VMEM((2 * KING, K), jnp.int32),
            pltpu.VMEM((K, D), jnp.float32),
            pltpu.VMEM((K, D), jnp.float32),
            pltpu.VMEM_SHARED((ACC_ROWS, D), jnp.float32),
            pltpu.SemaphoreType.DMA,
            pltpu.SemaphoreType.DMA,
            pltpu.SemaphoreType.DMA,
        ],
    )(_sc_scatter_body)


def _sc_scatter_body(g_hbm, srcp_hbm, dstp_hbm, out_hbm,
                     src_v, dst_ring, buf0, buf1, acc_sh, sem0, sem1, semi):
    c = lax.axis_index("c")
    s = lax.axis_index("s")
    w = c * NS + s

    zeros16 = jnp.zeros((16,), jnp.float32)

    def zbody(i, _):
        buf0[i // 8, pl.ds((i % 8) * 16, 16)] = zeros16
        return _

    lax.fori_loop(0, K * 8, zbody, None)
    base = s * ROWS_PER_TILE
    for kc in range(ROWS_PER_TILE // K):
        pltpu.sync_copy(buf0, acc_sh.at[pl.ds(base + kc * K, K)])

    pltpu.sync_copy(srcp_hbm.at[w], src_v)
    pltpu.sync_copy(dstp_hbm.at[w * NCH], dst_ring.at[pl.ds(0, KING)])
    pltpu.async_copy(dstp_hbm.at[w * NCH + 1], dst_ring.at[pl.ds(KING, KING)],
                     semi)
    pltpu.async_copy(g_hbm.at[src_v.at[0]], buf0, sem0)
    pltpu.async_copy(g_hbm.at[src_v.at[1]], buf1, sem1)
    plsc.subcore_barrier()

    def body(p, _):
        j = 2 * p
        pltpu.make_async_copy(g_hbm.at[src_v.at[j]], buf0, sem0).wait()
        pltpu.sync_copy(buf0, acc_sh.at[dst_ring.at[j % (2 * KING)]], add=True)
        pltpu.async_copy(g_hbm.at[src_v.at[j + 2]], buf0, sem0)
        pltpu.make_async_copy(g_hbm.at[src_v.at[j + 1]], buf1, sem1).wait()
        pltpu.sync_copy(buf1, acc_sh.at[dst_ring.at[(j + 1) % (2 * KING)]],
                        add=True)
        pltpu.async_copy(g_hbm.at[src_v.at[j + 3]], buf1, sem1)

        @pl.when(p % (KING // 2) == KING // 2 - 1)
        def _chunk():
            t = p // (KING // 2)
            pltpu.make_async_copy(
                dstp_hbm.at[w * NCH + t + 1],
                dst_ring.at[pl.ds(((t + 1) % 2) * KING, KING)], semi).wait()
            pltpu.async_copy(
                dstp_hbm.at[w * NCH + t + 2],
                dst_ring.at[pl.ds((t % 2) * KING, KING)], semi)

        return _

    lax.fori_loop(0, NB // 2, body, None)
    pltpu.make_async_copy(g_hbm.at[src_v.at[NB]], buf0, sem0).wait()
    pltpu.make_async_copy(g_hbm.at[src_v.at[NB + 1]], buf1, sem1).wait()
    pltpu.make_async_copy(dstp_hbm.at[w * NCH + NCH - 1],
                          dst_ring.at[pl.ds(((NCH - 1) % 2) * KING, KING)],
                          semi).wait()
    plsc.subcore_barrier()

    pltpu.sync_copy(acc_sh.at[pl.ds(base, ROWS_PER_TILE)], out_hbm.at[w])


_RB = 1000


def _tc_first_body(x_ref, w_ref, d0_ref, d1_ref, g_ref, dis_ref):
    dis = lax.rsqrt(d0_ref[...] + d1_ref[...] + 1.0)
    dis_ref[...] = dis
    g_ref[...] = jnp.dot(x_ref[...], w_ref[...],
                         preferred_element_type=jnp.float32) * dis


def _tc_first(x, W0, d0, d1):
    return pl.pallas_call(
        _tc_first_body,
        grid=(N_NODES // _RB,),
        in_specs=[
            pl.BlockSpec((_RB, D), lambda i: (i, 0)),
            pl.BlockSpec((D, D), lambda i: (0, 0)),
            pl.BlockSpec((_RB, 1), lambda i: (i, 0)),
            pl.BlockSpec((_RB, 1), lambda i: (i, 0)),
        ],
        out_specs=[
            pl.BlockSpec((_RB, D), lambda i: (i, 0)),
            pl.BlockSpec((_RB, 1), lambda i: (i, 0)),
        ],
        out_shape=[
            jax.ShapeDtypeStruct((N_NODES, D), jnp.float32),
            jax.ShapeDtypeStruct((N_NODES, 1), jnp.float32),
        ],
    )(x, W0, d0, d1)


def _tc_advance_body(a0_ref, a1_ref, g_ref, dis_ref, b_ref, w_ref, out_ref):
    dis = dis_ref[...]
    h = dis * (a0_ref[...] + a1_ref[...] + g_ref[...]) + b_ref[...]
    h = jnp.where(h >= 0, h, 0.1 * h)
    out_ref[...] = jnp.dot(h, w_ref[...],
                           preferred_element_type=jnp.float32) * dis


def _tc_advance(a0, a1, g, dis, b, W):
    return pl.pallas_call(
        _tc_advance_body,
        grid=(N_NODES // _RB,),
        in_specs=[
            pl.BlockSpec((_RB, D), lambda i: (i, 0)),
            pl.BlockSpec((_RB, D), lambda i: (i, 0)),
            pl.BlockSpec((_RB, D), lambda i: (i, 0)),
            pl.BlockSpec((_RB, 1), lambda i: (i, 0)),
            pl.BlockSpec((1, D), lambda i: (0, 0)),
            pl.BlockSpec((D, D), lambda i: (0, 0)),
        ],
        out_specs=pl.BlockSpec((_RB, D), lambda i: (i, 0)),
        out_shape=jax.ShapeDtypeStruct((N_NODES, D), jnp.float32),
    )(a0, a1, g, dis, b, W)


def _tc_final_body(a0_ref, a1_ref, u_ref, dis_ref, w_ref, out_ref):
    t = a0_ref[...] + a1_ref[...] + u_ref[...]
    out_ref[...] = jnp.dot(t, w_ref[...],
                           preferred_element_type=jnp.float32) * dis_ref[...]


def _tc_final(a0, a1, u, dis, Wp):
    return pl.pallas_call(
        _tc_final_body,
        grid=(N_NODES // _RB,),
        in_specs=[
            pl.BlockSpec((_RB, D), lambda i: (i, 0)),
            pl.BlockSpec((_RB, D), lambda i: (i, 0)),
            pl.BlockSpec((_RB, D), lambda i: (i, 0)),
            pl.BlockSpec((_RB, 1), lambda i: (i, 0)),
            pl.BlockSpec((D, D), lambda i: (0, 0)),
        ],
        out_specs=pl.BlockSpec((_RB, D), lambda i: (i, 0)),
        out_shape=jax.ShapeDtypeStruct((N_NODES, D), jnp.float32),
    )(a0, a1, u, dis, Wp)


def _acc_halves(a):
    a = a.reshape(NC, ACC_ROWS, D)
    return a[0, :N_NODES], a[1, :N_NODES]


def kernel(x, edge_index, Ws, bs, W_out, b_out):
    src = edge_index[0].astype(jnp.int32)
    dst = edge_index[1].astype(jnp.int32)
    npad = E_PAD - N_EDGES
    srcp = jnp.concatenate([src, jnp.zeros((npad,), jnp.int32)])
    dstp = jnp.concatenate(
        [dst, N_NODES + (jnp.arange(npad, dtype=jnp.int32) % (ACC_ROWS - N_NODES))])
    srcp = srcp.reshape(NW, NB, K)
    srcp = jnp.concatenate([srcp, jnp.zeros((NW, 2, K), jnp.int32)], axis=1)
    dstp = dstp.reshape(NW, NB, K)
    dstp = jnp.concatenate(
        [dstp, jnp.full((NW, 2 * KING, K), N_NODES, jnp.int32)], axis=1)
    dstp = dstp.reshape(NW * NCH, KING, K)

    scatter = _make_sc_scatter()

    d0, d1 = _acc_halves(scatter(jnp.ones((N_NODES, D), jnp.float32), srcp, dstp))
    g, dis = _tc_first(x, Ws[0], d0[:, 0:1], d1[:, 0:1])

    eye = jnp.eye(D, dtype=jnp.float32)
    for i in range(1, N_LAYERS + 1):
        a0, a1 = _acc_halves(scatter(g, srcp, dstp))
        W = Ws[i] if i < N_LAYERS else eye
        g = _tc_advance(a0, a1, g, dis, bs[i - 1][None, :], W)

    a0, a1 = _acc_halves(scatter(g, srcp, dstp))
    Wp = jnp.pad(W_out, ((0, 0), (0, D - 1)))
    o = _tc_final(a0, a1, g, dis, Wp)
    return o[:, 0] + b_out[0]


# speedup vs baseline: 1.4322x; 1.4322x over previous
import functools

import jax
import jax.numpy as jnp
from jax import lax
from jax.experimental import pallas as pl
from jax.experimental.pallas import tpu as pltpu
from jax.experimental.pallas import tpu_sc as plsc

N_NODES = 10000
N_EDGES = 320000
D = 128
N_LAYERS = 8

NC = 2
NS = 16
NW = NC * NS
K = 128
NB = 80
KING = 8
NCH = NB // KING + 2
EPT_PAD = NB * K
E_PAD = EPT_PAD * NW
ACC_ROWS = 10240
ROWS_PER_TILE = ACC_ROWS // NS


@functools.cache
def _make_sc_scatter():
    return functools.partial(
        pl.kernel,
        mesh=plsc.VectorSubcoreMesh(core_axis_name="c", subcore_axis_name="s"),
        out_type=jax.ShapeDtypeStruct((NW, ROWS_PER_TILE, D), jnp.float32),
        scratch_types=[
            pltpu.VMEM((NB + 2, K), jnp.int32),
            pltpu.VMEM((2 * KING, K), jnp.int32),
            pltpu.VMEM((K, D), jnp.float32),
            pltpu.VMEM((K, D), jnp.float32),
            pltpu.VMEM_SHARED((ACC_ROWS, D), jnp.float32),
            pltpu.SemaphoreType.DMA,
            pltpu.SemaphoreType.DMA,
            pltpu.SemaphoreType.DMA,
        ],
    )(_sc_scatter_body)


def _sc_scatter_body(g_hbm, srcp_hbm, dstp_hbm, out_hbm,
                     src_v, dst_ring, buf0, buf1, acc_sh, sem0, sem1, semi):
    c = lax.axis_index("c")
    s = lax.axis_index("s")
    w = c * NS + s

    zeros16 = jnp.zeros((16,), jnp.float32)

    def zbody(i, _):
        buf0[i // 8, pl.ds((i % 8) * 16, 16)] = zeros16
        return _

    lax.fori_loop(0, K * 8, zbody, None)
    base = s * ROWS_PER_TILE
    for kc in range(ROWS_PER_TILE // K):
        pltpu.sync_copy(buf0, acc_sh.at[pl.ds(base + kc * K, K)])

    pltpu.sync_copy(srcp_hbm.at[w], src_v)
    pltpu.sync_copy(dstp_hbm.at[w * NCH], dst_ring.at[pl.ds(0, KING)])
    pltpu.async_copy(dstp_hbm.at[w * NCH + 1], dst_ring.at[pl.ds(KING, KING)],
                     semi)
    plsc.subcore_barrier()

    def body(p, _):
        j = 2 * p
        pltpu.sync_copy(g_hbm.at[src_v.at[j]], buf0)
        pltpu.sync_copy(buf0, acc_sh.at[dst_ring.at[j % (2 * KING)]], add=True)
        pltpu.sync_copy(g_hbm.at[src_v.at[j + 1]], buf1)
        pltpu.sync_copy(buf1, acc_sh.at[dst_ring.at[(j + 1) % (2 * KING)]],
                        add=True)

        @pl.when(p % (KING // 2) == KING // 2 - 1)
        def _chunk():
            t = p // (KING // 2)
            pltpu.make_async_copy(
                dstp_hbm.at[w * NCH + t + 1],
                dst_ring.at[pl.ds(((t + 1) % 2) * KING, KING)], semi).wait()
            pltpu.async_copy(
                dstp_hbm.at[w * NCH + t + 2],
                dst_ring.at[pl.ds((t % 2) * KING, KING)], semi)

        return _

    lax.fori_loop(0, NB // 2, body, None)
    pltpu.make_async_copy(dstp_hbm.at[w * NCH + NCH - 1],
                          dst_ring.at[pl.ds(((NCH - 1) % 2) * KING, KING)],
                          semi).wait()
    plsc.subcore_barrier()

    pltpu.sync_copy(acc_sh.at[pl.ds(base, ROWS_PER_TILE)], out_hbm.at[w])


_RB = 1000


def _tc_first_body(x_ref, w_ref, d0_ref, d1_ref, g_ref, dis_ref):
    dis = lax.rsqrt(d0_ref[...] + d1_ref[...] + 1.0)
    dis_ref[...] = dis
    g_ref[...] = jnp.dot(x_ref[...], w_ref[...],
                         preferred_element_type=jnp.float32) * dis


def _tc_first(x, W0, d0, d1):
    return pl.pallas_call(
        _tc_first_body,
        grid=(N_NODES // _RB,),
        in_specs=[
            pl.BlockSpec((_RB, D), lambda i: (i, 0)),
            pl.BlockSpec((D, D), lambda i: (0, 0)),
            pl.BlockSpec((_RB, 1), lambda i: (i, 0)),
            pl.BlockSpec((_RB, 1), lambda i: (i, 0)),
        ],
        out_specs=[
            pl.BlockSpec((_RB, D), lambda i: (i, 0)),
            pl.BlockSpec((_RB, 1), lambda i: (i, 0)),
        ],
        out_shape=[
            jax.ShapeDtypeStruct((N_NODES, D), jnp.float32),
            jax.ShapeDtypeStruct((N_NODES, 1), jnp.float32),
        ],
    )(x, W0, d0, d1)


def _tc_advance_body(a0_ref, a1_ref, g_ref, dis_ref, b_ref, w_ref, out_ref):
    dis = dis_ref[...]
    h = dis * (a0_ref[...] + a1_ref[...] + g_ref[...]) + b_ref[...]
    h = jnp.where(h >= 0, h, 0.1 * h)
    out_ref[...] = jnp.dot(h, w_ref[...],
                           preferred_element_type=jnp.float32) * dis


def _tc_advance(a0, a1, g, dis, b, W):
    return pl.pallas_call(
        _tc_advance_body,
        grid=(N_NODES // _RB,),
        in_specs=[
            pl.BlockSpec((_RB, D), lambda i: (i, 0)),
            pl.BlockSpec((_RB, D), lambda i: (i, 0)),
            pl.BlockSpec((_RB, D), lambda i: (i, 0)),
            pl.BlockSpec((_RB, 1), lambda i: (i, 0)),
            pl.BlockSpec((1, D), lambda i: (0, 0)),
            pl.BlockSpec((D, D), lambda i: (0, 0)),
        ],
        out_specs=pl.BlockSpec((_RB, D), lambda i: (i, 0)),
        out_shape=jax.ShapeDtypeStruct((N_NODES, D), jnp.float32),
    )(a0, a1, g, dis, b, W)


def _tc_final_body(a0_ref, a1_ref, u_ref, dis_ref, w_ref, out_ref):
    t = a0_ref[...] + a1_ref[...] + u_ref[...]
    out_ref[...] = jnp.dot(t, w_ref[...],
                           preferred_element_type=jnp.float32) * dis_ref[...]


def _tc_final(a0, a1, u, dis, Wp):
    return pl.pallas_call(
        _tc_final_body,
        grid=(N_NODES // _RB,),
        in_specs=[
            pl.BlockSpec((_RB, D), lambda i: (i, 0)),
            pl.BlockSpec((_RB, D), lambda i: (i, 0)),
            pl.BlockSpec((_RB, D), lambda i: (i, 0)),
            pl.BlockSpec((_RB, 1), lambda i: (i, 0)),
            pl.BlockSpec((D, D), lambda i: (0, 0)),
        ],
        out_specs=pl.BlockSpec((_RB, D), lambda i: (i, 0)),
        out_shape=jax.ShapeDtypeStruct((N_NODES, D), jnp.float32),
    )(a0, a1, u, dis, Wp)


def _acc_halves(a):
    a = a.reshape(NC, ACC_ROWS, D)
    return a[0, :N_NODES], a[1, :N_NODES]


def kernel(x, edge_index, Ws, bs, W_out, b_out):
    src = edge_index[0].astype(jnp.int32)
    dst = edge_index[1].astype(jnp.int32)
    npad = E_PAD - N_EDGES
    srcp = jnp.concatenate([src, jnp.zeros((npad,), jnp.int32)])
    dstp = jnp.concatenate(
        [dst, N_NODES + (jnp.arange(npad, dtype=jnp.int32) % (ACC_ROWS - N_NODES))])
    srcp = srcp.reshape(NW, NB, K)
    srcp = jnp.concatenate([srcp, jnp.zeros((NW, 2, K), jnp.int32)], axis=1)
    dstp = dstp.reshape(NW, NB, K)
    dstp = jnp.concatenate(
        [dstp, jnp.full((NW, 2 * KING, K), N_NODES, jnp.int32)], axis=1)
    dstp = dstp.reshape(NW * NCH, KING, K)

    scatter = _make_sc_scatter()

    d0, d1 = _acc_halves(scatter(jnp.ones((N_NODES, D), jnp.float32), srcp, dstp))
    g, dis = _tc_first(x, Ws[0], d0[:, 0:1], d1[:, 0:1])

    eye = jnp.eye(D, dtype=jnp.float32)
    for i in range(1, N_LAYERS + 1):
        a0, a1 = _acc_halves(scatter(g, srcp, dstp))
        W = Ws[i] if i < N_LAYERS else eye
        g = _tc_advance(a0, a1, g, dis, bs[i - 1][None, :], W)

    a0, a1 = _acc_halves(scatter(g, srcp, dstp))
    Wp = jnp.pad(W_out, ((0, 0), (0, D - 1)))
    o = _tc_final(a0, a1, g, dis, Wp)
    return o[:, 0] + b_out[0]


# speedup vs baseline: 1.5333x; 1.0706x over previous
import functools

import jax
import jax.numpy as jnp
from jax import lax
from jax.experimental import pallas as pl
from jax.experimental.pallas import tpu as pltpu
from jax.experimental.pallas import tpu_sc as plsc

N_NODES = 10000
N_EDGES = 320000
D = 128
N_LAYERS = 8

NC = 2
NS = 16
NW = NC * NS
K = 128
NB = 80
KING = 8
NCH = NB // KING
EPT_PAD = NB * K
E_PAD = EPT_PAD * NW
ACC_ROWS = 10240
ROWS_PER_TILE = ACC_ROWS // NS


@functools.cache
def _make_sc_scatter():
    return functools.partial(
        pl.kernel,
        mesh=plsc.VectorSubcoreMesh(core_axis_name="c", subcore_axis_name="s"),
        out_type=jax.ShapeDtypeStruct((NW, ROWS_PER_TILE, D), jnp.float32),
        scratch_types=[
            pltpu.VMEM((NB, K), jnp.int32),
            pltpu.VMEM((2 * KING, K), jnp.int32),
            pltpu.VMEM((K, D), jnp.float32),
            pltpu.VMEM((K, D), jnp.float32),
            pltpu.VMEM_SHARED((ACC_ROWS, D), jnp.float32),
            pltpu.SemaphoreType.DMA,
            pltpu.SemaphoreType.DMA,
            pltpu.SemaphoreType.DMA,
        ],
    )(_sc_scatter_body)


def _sc_scatter_body(g_hbm, srcp_hbm, dstp_hbm, out_hbm,
                     src_v, dst_ring, buf0, buf1, acc_sh, sem0, sem1, semi):
    c = lax.axis_index("c")
    s = lax.axis_index("s")
    w = c * NS + s

    zeros16 = jnp.zeros((16,), jnp.float32)

    def zbody(i, _):
        buf0[i // 8, pl.ds((i % 8) * 16, 16)] = zeros16
        return _

    lax.fori_loop(0, K * 8, zbody, None)
    base = s * ROWS_PER_TILE
    for kc in range(ROWS_PER_TILE // K):
        pltpu.sync_copy(buf0, acc_sh.at[pl.ds(base + kc * K, K)])

    pltpu.sync_copy(srcp_hbm.at[w], src_v)
    pltpu.sync_copy(dstp_hbm.at[w * NCH], dst_ring.at[pl.ds(0, KING)])
    pltpu.async_copy(dstp_hbm.at[w * NCH + 1], dst_ring.at[pl.ds(KING, KING)],
                     semi)
    plsc.subcore_barrier()

    for t in range(NCH):
        half = (t % 2) * KING
        jbase = t * KING

        def body(q, _, half=half, jbase=jbase):
            j = jbase + 2 * q
            r = half + 2 * q
            pltpu.sync_copy(g_hbm.at[src_v.at[j]], buf0)
            pltpu.sync_copy(buf0, acc_sh.at[dst_ring.at[r]], add=True)
            pltpu.sync_copy(g_hbm.at[src_v.at[j + 1]], buf1)
            pltpu.sync_copy(buf1, acc_sh.at[dst_ring.at[r + 1]], add=True)
            return _

        lax.fori_loop(0, KING // 2, body, None)
        if t + 1 < NCH:
            pltpu.make_async_copy(
                dstp_hbm.at[w * NCH + t + 1],
                dst_ring.at[pl.ds(((t + 1) % 2) * KING, KING)], semi).wait()
        if t + 2 < NCH:
            pltpu.async_copy(dstp_hbm.at[w * NCH + t + 2],
                             dst_ring.at[pl.ds(half, KING)], semi)
    plsc.subcore_barrier()

    pltpu.sync_copy(acc_sh.at[pl.ds(base, ROWS_PER_TILE)], out_hbm.at[w])


_RB = 1000


def _tc_first_body(x_ref, w_ref, d0_ref, d1_ref, g_ref, dis_ref):
    dis = lax.rsqrt(d0_ref[...] + d1_ref[...] + 1.0)
    dis_ref[...] = dis
    g_ref[...] = jnp.dot(x_ref[...], w_ref[...],
                         preferred_element_type=jnp.float32) * dis


def _tc_first(x, W0, d0, d1):
    return pl.pallas_call(
        _tc_first_body,
        grid=(N_NODES // _RB,),
        in_specs=[
            pl.BlockSpec((_RB, D), lambda i: (i, 0)),
            pl.BlockSpec((D, D), lambda i: (0, 0)),
            pl.BlockSpec((_RB, 1), lambda i: (i, 0)),
            pl.BlockSpec((_RB, 1), lambda i: (i, 0)),
        ],
        out_specs=[
            pl.BlockSpec((_RB, D), lambda i: (i, 0)),
            pl.BlockSpec((_RB, 1), lambda i: (i, 0)),
        ],
        out_shape=[
            jax.ShapeDtypeStruct((N_NODES, D), jnp.float32),
            jax.ShapeDtypeStruct((N_NODES, 1), jnp.float32),
        ],
    )(x, W0, d0, d1)


def _tc_advance_body(a0_ref, a1_ref, g_ref, dis_ref, b_ref, w_ref, out_ref):
    dis = dis_ref[...]
    h = dis * (a0_ref[...] + a1_ref[...] + g_ref[...]) + b_ref[...]
    h = jnp.where(h >= 0, h, 0.1 * h)
    out_ref[...] = jnp.dot(h, w_ref[...],
                           preferred_element_type=jnp.float32) * dis


def _tc_advance(a0, a1, g, dis, b, W):
    return pl.pallas_call(
        _tc_advance_body,
        grid=(N_NODES // _RB,),
        in_specs=[
            pl.BlockSpec((_RB, D), lambda i: (i, 0)),
            pl.BlockSpec((_RB, D), lambda i: (i, 0)),
            pl.BlockSpec((_RB, D), lambda i: (i, 0)),
            pl.BlockSpec((_RB, 1), lambda i: (i, 0)),
            pl.BlockSpec((1, D), lambda i: (0, 0)),
            pl.BlockSpec((D, D), lambda i: (0, 0)),
        ],
        out_specs=pl.BlockSpec((_RB, D), lambda i: (i, 0)),
        out_shape=jax.ShapeDtypeStruct((N_NODES, D), jnp.float32),
    )(a0, a1, g, dis, b, W)


def _tc_final_body(a0_ref, a1_ref, u_ref, dis_ref, w_ref, out_ref):
    t = a0_ref[...] + a1_ref[...] + u_ref[...]
    out_ref[...] = jnp.dot(t, w_ref[...],
                           preferred_element_type=jnp.float32) * dis_ref[...]


def _tc_final(a0, a1, u, dis, Wp):
    return pl.pallas_call(
        _tc_final_body,
        grid=(N_NODES // _RB,),
        in_specs=[
            pl.BlockSpec((_RB, D), lambda i: (i, 0)),
            pl.BlockSpec((_RB, D), lambda i: (i, 0)),
            pl.BlockSpec((_RB, D), lambda i: (i, 0)),
            pl.BlockSpec((_RB, 1), lambda i: (i, 0)),
            pl.BlockSpec((D, D), lambda i: (0, 0)),
        ],
        out_specs=pl.BlockSpec((_RB, D), lambda i: (i, 0)),
        out_shape=jax.ShapeDtypeStruct((N_NODES, D), jnp.float32),
    )(a0, a1, u, dis, Wp)


def _acc_halves(a):
    a = a.reshape(NC, ACC_ROWS, D)
    return a[0, :N_NODES], a[1, :N_NODES]


def kernel(x, edge_index, Ws, bs, W_out, b_out):
    src = edge_index[0].astype(jnp.int32)
    dst = edge_index[1].astype(jnp.int32)
    npad = E_PAD - N_EDGES
    srcp = jnp.concatenate([src, jnp.zeros((npad,), jnp.int32)])
    dstp = jnp.concatenate(
        [dst, N_NODES + (jnp.arange(npad, dtype=jnp.int32) % (ACC_ROWS - N_NODES))])
    srcp = srcp.reshape(NW, NB, K)
    dstp = dstp.reshape(NW * NCH, KING, K)

    scatter = _make_sc_scatter()

    d0, d1 = _acc_halves(scatter(jnp.ones((N_NODES, D), jnp.float32), srcp, dstp))
    g, dis = _tc_first(x, Ws[0], d0[:, 0:1], d1[:, 0:1])

    eye = jnp.eye(D, dtype=jnp.float32)
    for i in range(1, N_LAYERS + 1):
        a0, a1 = _acc_halves(scatter(g, srcp, dstp))
        W = Ws[i] if i < N_LAYERS else eye
        g = _tc_advance(a0, a1, g, dis, bs[i - 1][None, :], W)

    a0, a1 = _acc_halves(scatter(g, srcp, dstp))
    Wp = jnp.pad(W_out, ((0, 0), (0, D - 1)))
    o = _tc_final(a0, a1, g, dis, Wp)
    return o[:, 0] + b_out[0]
